# baseline (speedup 1.0000x reference)
import functools

import jax
import jax.numpy as jnp
from jax import lax
from jax.experimental import pallas as pl
from jax.experimental.pallas import tpu as pltpu
from jax.experimental.pallas import tpu_sc as plsc

_R = 2
_L = 16
_W = 128
_UNROLL = 1


def _take(v, idx):
    return v.at[idx].get(mode="promise_in_bounds")


@functools.lru_cache(maxsize=None)
def _build(B, S, K, E, D):
    LE = E // D
    NC, NS = 2, 16
    NW = NC * NS
    H = NW // (D * B)
    NBLK = S // _W
    CBLK = NBLK // H
    RBLK = CBLK // _R
    mesh = plsc.VectorSubcoreMesh(core_axis_name="c", subcore_axis_name="s")

    @functools.partial(
        pl.kernel,
        out_type=(
            jax.ShapeDtypeStruct((D * B, NBLK * LE, _W), jnp.float32),
            jax.ShapeDtypeStruct((D * B, NBLK * LE // _R, _W), jnp.float32),
        ),
        mesh=mesh,
        compiler_params=pltpu.CompilerParams(needs_layout_passes=False),
        scratch_types=(
            pltpu.VMEM((CBLK * K, _W), jnp.float32),
            pltpu.VMEM((CBLK * K, _W), jnp.int32),
            pltpu.VMEM((E,), jnp.int32),
            pltpu.VMEM((E,), jnp.int32),
            pltpu.VMEM((CBLK * LE, _W), jnp.float32),
            pltpu.VMEM((RBLK * LE, _W), jnp.float32),
            pltpu.SemaphoreType.DMA,
            pltpu.SemaphoreType.DMA,
        ),
    )
    def launch(topk_hbm, meta_hbm, dev_hbm, out_hbm, red_hbm,
               w_v, m_v, dev_v, le_v, out_v, red_v, in_sem, out_sem):
        wid = lax.axis_index("s") * NC + lax.axis_index("c")
        d = wid // (B * H)
        b = (wid // H) % B
        h = wid % H

        cp_le = pltpu.make_async_copy(dev_hbm, dev_v, in_sem)
        cp_w = pltpu.make_async_copy(
            topk_hbm.at[b, pl.ds(h * CBLK * K, CBLK * K), :], w_v, in_sem)
        cp_m = pltpu.make_async_copy(
            meta_hbm.at[b, pl.ds(h * CBLK * K, CBLK * K), :], m_v, in_sem)
        cp_le.start()
        cp_w.start()
        cp_m.start()
        cp_le.wait()

        iota = lax.broadcasted_iota(jnp.int32, (_L,), 0)

        dev = plsc.load_gather(dev_v, [iota])
        rank = jnp.zeros((_L,), jnp.int32)
        for dd in range(D):
            on = (dev == dd).astype(jnp.int32)
            c = plsc.cumsum(on) - on
            rank = jnp.where(dev == dd, c, rank)
        plsc.store_scatter(le_v, [dev * LE + rank], iota)

        evs = [
            plsc.load_gather(le_v, [jnp.broadcast_to(d * LE + j, (_L,))])
            for j in range(LE)
        ]
        zero = jnp.zeros((_L,), jnp.float32)
        idx_e = (2 * iota) & (_L - 1)
        lo = iota < (_L // 2)
        swp = iota ^ 1

        cp_w.wait()
        cp_m.wait()

        def body(g, carry):
            for bu in range(_UNROLL):
                blk = g * _UNROLL + bu
                ms = [[m_v[blk * K + k, pl.ds(u * _L, _L)]
                       for u in range(_W // _L)] for k in range(K)]
                ws = [[w_v[blk * K + k, pl.ds(u * _L, _L)]
                       for u in range(_W // _L)] for k in range(K)]
                for j in range(LE):
                    os_ = []
                    for u in range(_W // _L):
                        o = zero
                        for k in range(K):
                            o = o + jnp.where(ms[k][u] == evs[j], ws[k][u], zero)
                        out_v[blk * LE + j, pl.ds(u * _L, _L)] = o
                        os_.append(o)
                    rrow = (blk // _R) * LE + j
                    rcol = (blk % _R) * (_W // _R)
                    for t in range(_W // _L // _R):
                        ma = jnp.maximum(os_[2 * t], _take(os_[2 * t], swp))
                        mc = jnp.maximum(os_[2 * t + 1],
                                         _take(os_[2 * t + 1], swp))
                        red_v[rrow, pl.ds(rcol + t * _L, _L)] = jnp.where(
                            lo, _take(ma, idx_e), _take(mc, idx_e))
            return carry

        lax.fori_loop(0, CBLK // _UNROLL, body, 0)

        cp_out = pltpu.make_async_copy(
            out_v, out_hbm.at[d * B + b, pl.ds(h * CBLK * LE, CBLK * LE), :],
            out_sem)
        cp_red = pltpu.make_async_copy(
            red_v, red_hbm.at[d * B + b, pl.ds(h * RBLK * LE, RBLK * LE), :],
            out_sem)
        cp_out.start()
        cp_red.start()
        cp_out.wait()
        cp_red.wait()

    return launch


def kernel(topk_tensor, expert_mapping, expert_metadata):
    _, B, S, K = topk_tensor.shape
    E, D = expert_mapping.shape[2], expert_mapping.shape[3]
    LE = E // D
    NBLK = S // _W

    def to_lanes(x):
        return (x.reshape(B, NBLK, _W, K)
                 .transpose(0, 1, 3, 2)
                 .reshape(B, NBLK * K, _W))

    dev = jnp.argmax(expert_mapping.reshape(E, D), axis=1).astype(jnp.int32)

    launch = _build(B, S, K, E, D)
    out3, red3 = launch(
        to_lanes(topk_tensor),
        to_lanes(expert_metadata),
        dev,
    )
    out = (out3.reshape(D, B, NBLK, LE, _W)
               .transpose(0, 1, 2, 4, 3)
               .reshape(D, B, S, LE))
    red = (red3.reshape(D, B, NBLK // _R, LE, _W)
               .transpose(0, 1, 2, 4, 3)
               .reshape(D, B, S // _R, LE))
    return out, red


# speedup vs baseline: 1.0034x; 1.0034x over previous
import functools

import jax
import jax.numpy as jnp
from jax import lax
from jax.experimental import pallas as pl
from jax.experimental.pallas import tpu as pltpu
from jax.experimental.pallas import tpu_sc as plsc

_R = 2
_L = 16
_W = 128
_UNROLL = 1


def _take(v, idx):
    return v.at[idx].get(mode="promise_in_bounds")


@functools.lru_cache(maxsize=None)
def _build(B, S, K, E, D):
    LE = E // D
    NC, NS = 2, 16
    NW = NC * NS
    H = NW // (D * B)
    NBLK = S // _W
    CBLK = NBLK // H
    RBLK = CBLK // _R
    mesh = plsc.VectorSubcoreMesh(core_axis_name="c", subcore_axis_name="s")

    @functools.partial(
        pl.kernel,
        out_type=(
            jax.ShapeDtypeStruct((D * B, NBLK * LE, _W), jnp.float32),
            jax.ShapeDtypeStruct((D * B, NBLK * LE // _R, _W), jnp.float32),
        ),
        mesh=mesh,
        compiler_params=pltpu.CompilerParams(needs_layout_passes=False),
        scratch_types=(
            pltpu.VMEM((CBLK * K, _W), jnp.float32),
            pltpu.VMEM((CBLK * K, _W), jnp.int32),
            pltpu.VMEM((E,), jnp.int32),
            pltpu.VMEM((CBLK * LE, _W), jnp.float32),
            pltpu.VMEM((RBLK * LE, _W), jnp.float32),
            pltpu.SemaphoreType.DMA,
            pltpu.SemaphoreType.DMA,
        ),
    )
    def launch(topk_hbm, meta_hbm, le_hbm, out_hbm, red_hbm,
               w_v, m_v, le_v, out_v, red_v, in_sem, out_sem):
        wid = lax.axis_index("s") * NC + lax.axis_index("c")
        d = wid // (B * H)
        b = (wid // H) % B
        h = wid % H

        cp_le = pltpu.make_async_copy(le_hbm, le_v, in_sem)
        cp_w = pltpu.make_async_copy(
            topk_hbm.at[b, pl.ds(h * CBLK * K, CBLK * K), :], w_v, in_sem)
        cp_m = pltpu.make_async_copy(
            meta_hbm.at[b, pl.ds(h * CBLK * K, CBLK * K), :], m_v, in_sem)
        cp_le.start()
        cp_w.start()
        cp_m.start()
        cp_le.wait()

        iota = lax.broadcasted_iota(jnp.int32, (_L,), 0)
        evs = [
            plsc.load_gather(le_v, [jnp.broadcast_to(d * LE + j, (_L,))])
            for j in range(LE)
        ]
        zero = jnp.zeros((_L,), jnp.float32)
        idx_e = (2 * iota) & (_L - 1)
        lo = iota < (_L // 2)
        swp = iota ^ 1

        cp_w.wait()
        cp_m.wait()

        def body(g, carry):
            for bu in range(_UNROLL):
                blk = g * _UNROLL + bu
                ms = [[m_v[blk * K + k, pl.ds(u * _L, _L)]
                       for u in range(_W // _L)] for k in range(K)]
                ws = [[w_v[blk * K + k, pl.ds(u * _L, _L)]
                       for u in range(_W // _L)] for k in range(K)]
                for j in range(LE):
                    os_ = []
                    for u in range(_W // _L):
                        o = zero
                        for k in range(K):
                            o = o + jnp.where(ms[k][u] == evs[j], ws[k][u], zero)
                        out_v[blk * LE + j, pl.ds(u * _L, _L)] = o
                        os_.append(o)
                    rrow = (blk // _R) * LE + j
                    rcol = (blk % _R) * (_W // _R)
                    for t in range(_W // _L // _R):
                        ma = jnp.maximum(os_[2 * t], _take(os_[2 * t], swp))
                        mc = jnp.maximum(os_[2 * t + 1],
                                         _take(os_[2 * t + 1], swp))
                        red_v[rrow, pl.ds(rcol + t * _L, _L)] = jnp.where(
                            lo, _take(ma, idx_e), _take(mc, idx_e))
            return carry

        lax.fori_loop(0, CBLK // _UNROLL, body, 0)

        cp_out = pltpu.make_async_copy(
            out_v, out_hbm.at[d * B + b, pl.ds(h * CBLK * LE, CBLK * LE), :],
            out_sem)
        cp_red = pltpu.make_async_copy(
            red_v, red_hbm.at[d * B + b, pl.ds(h * RBLK * LE, RBLK * LE), :],
            out_sem)
        cp_out.start()
        cp_red.start()
        cp_out.wait()
        cp_red.wait()

    return launch


def kernel(topk_tensor, expert_mapping, expert_metadata):
    _, B, S, K = topk_tensor.shape
    E, D = expert_mapping.shape[2], expert_mapping.shape[3]
    LE = E // D
    NBLK = S // _W

    def to_lanes(x):
        return (x.reshape(B, NBLK, _W, K)
                 .transpose(0, 1, 3, 2)
                 .reshape(B, NBLK * K, _W))

    mapping = expert_mapping.reshape(E, D)
    dev = jnp.argmax(mapping, axis=1).astype(jnp.int32)
    same = (dev[:, None] == dev[None, :]).astype(jnp.int32)
    tril = (jnp.arange(E)[:, None] > jnp.arange(E)[None, :]).astype(jnp.int32)
    rank = jnp.sum(same * tril, axis=1, dtype=jnp.int32)
    pos = dev * LE + rank
    le = jnp.sum(
        jnp.where(pos[:, None] == jnp.arange(E)[None, :],
                  jnp.arange(E, dtype=jnp.int32)[:, None], 0),
        axis=0, dtype=jnp.int32)

    launch = _build(B, S, K, E, D)
    out3, red3 = launch(
        to_lanes(topk_tensor),
        to_lanes(expert_metadata),
        le,
    )
    out = (out3.reshape(D, B, NBLK, LE, _W)
               .transpose(0, 1, 2, 4, 3)
               .reshape(D, B, S, LE))
    red = (red3.reshape(D, B, NBLK // _R, LE, _W)
               .transpose(0, 1, 2, 4, 3)
               .reshape(D, B, S // _R, LE))
    return out, red
